# baseline (speedup 1.0000x reference)
import functools

import jax
import jax.numpy as jnp
from jax import lax
from jax.experimental import pallas as pl
from jax.experimental.pallas import tpu as pltpu
from jax.experimental.pallas import tpu_sc as plsc

USER_NUM = 1000000
EMBED_DIM = 64
BATCH = 16384

_NC = 2
_NS = 16
_NW = _NC * _NS
_B_PER_W = BATCH // _NW


def _gather_body(table_hbm, idx_hbm, out_hbm, idx_v, rows_v, sem):
    wid = lax.axis_index("s") * _NC + lax.axis_index("c")
    base = wid * _B_PER_W
    pltpu.sync_copy(idx_hbm.at[pl.ds(base, _B_PER_W)], idx_v)
    pltpu.async_copy(table_hbm.at[idx_v], rows_v, sem).wait()
    pltpu.sync_copy(rows_v, out_hbm.at[pl.ds(base, _B_PER_W)])


@jax.jit
def kernel(nodes, embedding_weight):
    mesh = plsc.VectorSubcoreMesh(core_axis_name="c", subcore_axis_name="s")
    run = functools.partial(
        pl.kernel,
        mesh=mesh,
        out_type=jax.ShapeDtypeStruct((BATCH, EMBED_DIM), jnp.float32),
        scratch_types=[
            pltpu.VMEM((_B_PER_W,), jnp.int32),
            pltpu.VMEM((_B_PER_W, EMBED_DIM), jnp.float32),
            pltpu.SemaphoreType.DMA,
        ],
        compiler_params=pltpu.CompilerParams(use_tc_tiling_on_sc=False),
    )(_gather_body)
    return run(embedding_weight, nodes.astype(jnp.int32))


# speedup vs baseline: 2.9703x; 2.9703x over previous
import functools

import jax
import jax.numpy as jnp
from jax import lax
from jax.experimental import pallas as pl
from jax.experimental.pallas import tpu as pltpu
from jax.experimental.pallas import tpu_sc as plsc

USER_NUM = 1000000
EMBED_DIM = 64
BATCH = 16384

_NC = 2
_NS = 16
_NW = _NC * _NS
_B_PER_W = BATCH // _NW
_NBUF = 8
_GRP = 16
_NGRP = _B_PER_W // _GRP
_LANE = 16


def _issue(table_t, buf_v, sem, c, slot):
    cb = pl.multiple_of((c >> 7) << 7, 128)
    return pltpu.async_copy(
        table_t.at[:, pl.ds(cb, 128)], buf_v.at[slot], sem
    )


def _wait(table_t, buf_v, sem, slot):
    pltpu.make_async_copy(
        table_t.at[:, pl.ds(0, 128)], buf_v.at[slot], sem
    ).wait()


def _gather_body(table_t, idx_hbm, out_hbm, idx_v, buf_v, out_v, sem):
    wid = lax.axis_index("s") * _NC + lax.axis_index("c")
    base = wid * _B_PER_W
    pltpu.sync_copy(idx_hbm.at[pl.ds(base, _B_PER_W)], idx_v)

    vec0 = idx_v[pl.ds(0, _LANE)]
    for b in range(_NBUF):
        _issue(table_t, buf_v, sem, vec0[b], b)

    rows16 = lax.iota(jnp.int32, _LANE)

    def group(g, carry):
        cur = idx_v[pl.ds(g * _GRP, _LANE)]
        gnext = jnp.minimum(g + 1, _NGRP - 1)
        nxt = idx_v[pl.ds(gnext * _GRP, _LANE)]
        for b in range(_GRP):
            j = g * _GRP + b
            slot = b % _NBUF
            _wait(table_t, buf_v, sem, slot)
            r = cur[b] & 127
            col16 = jnp.full((_LANE,), r, jnp.int32)
            for k in range(EMBED_DIM // _LANE):
                piece = plsc.load_gather(
                    buf_v.at[slot], [rows16 + k * _LANE, col16]
                )
                out_v[pl.ds(j * EMBED_DIM + k * _LANE, _LANE)] = piece
            cnext = cur[b + _NBUF] if b < _GRP - _NBUF else nxt[b - (_GRP - _NBUF)]

            @pl.when(j + _NBUF < _B_PER_W)
            def _():
                _issue(table_t, buf_v, sem, cnext, slot)

        return carry

    lax.fori_loop(0, _NGRP, group, 0)
    pltpu.sync_copy(out_v, out_hbm.at[pl.ds(base * EMBED_DIM, _B_PER_W * EMBED_DIM)])


@jax.jit
def kernel(nodes, embedding_weight):
    mesh = plsc.VectorSubcoreMesh(core_axis_name="c", subcore_axis_name="s")
    run = functools.partial(
        pl.kernel,
        mesh=mesh,
        out_type=jax.ShapeDtypeStruct((BATCH * EMBED_DIM,), jnp.float32),
        scratch_types=[
            pltpu.VMEM((_B_PER_W,), jnp.int32),
            pltpu.VMEM((_NBUF, EMBED_DIM, 128), jnp.float32),
            pltpu.VMEM((_B_PER_W * EMBED_DIM,), jnp.float32),
            pltpu.SemaphoreType.DMA,
        ],
        compiler_params=pltpu.CompilerParams(
            use_tc_tiling_on_sc=True, needs_layout_passes=False
        ),
    )(_gather_body)
    flat = run(embedding_weight.T, nodes.astype(jnp.int32))
    return flat.reshape(BATCH, EMBED_DIM)


# speedup vs baseline: 3.1589x; 1.0635x over previous
import functools

import jax
import jax.numpy as jnp
from jax import lax
from jax.experimental import pallas as pl
from jax.experimental.pallas import tpu as pltpu
from jax.experimental.pallas import tpu_sc as plsc

USER_NUM = 1000000
EMBED_DIM = 64
BATCH = 16384

_NC = 2
_NS = 16
_NW = _NC * _NS
_LANE = 16
_CCOLS = 256
_NCHUNK = -(-USER_NUM // _CCOLS)
_CPW = 124
_MAXOFF = USER_NUM - 192
_NSLOT = 4
_BROWS = 64
_DUMP = BATCH


def _issue(table_t, cbuf, sem, g, slot):
    coff = pl.multiple_of(jnp.minimum(g * _CCOLS, _MAXOFF), 128)
    return pltpu.async_copy(
        table_t.at[:, pl.ds(coff, _CCOLS)], cbuf.at[slot], sem
    )


def _wait_chunk(table_t, cbuf, sem, slot):
    pltpu.make_async_copy(
        table_t.at[:, pl.ds(0, _CCOLS)], cbuf.at[slot], sem
    ).wait()


def _body(table_t, idx_hbm, out_hbm, idx_all, clist, plist, cbuf, batch,
          posb, sem_c, sem_s):
    wid = lax.axis_index("s") * _NC + lax.axis_index("c")
    wlo = wid * _CPW
    whi = jnp.minimum(wlo + _CPW, _NCHUNK)
    clo = wlo * _CCOLS
    chi = jnp.minimum(whi * _CCOLS, USER_NUM)

    pltpu.sync_copy(idx_hbm, idx_all)

    iota = lax.iota(jnp.int32, _LANE)

    def scan_in(t, cnt):
        v = idx_all[pl.ds(t * _LANE, _LANE)]
        m = (v >= clo) & (v < chi)
        ps = plsc.cumsum(m.astype(jnp.int32))
        tgt = cnt + ps - 1
        plsc.store_scatter(clist, [tgt], v, mask=m)
        plsc.store_scatter(plist, [tgt], t * _LANE + iota, mask=m)
        return cnt + plsc.all_reduce_population_count(m)[0]

    cnt = lax.fori_loop(0, BATCH // _LANE, scan_in, jnp.int32(0))
    nvec = (cnt + _LANE - 1) // _LANE

    for k in range(_BROWS // _LANE):
        posb[pl.ds(k * _LANE, _LANE)] = jnp.full((_LANE,), _DUMP, jnp.int32)

    def flush():
        pltpu.async_copy(batch, out_hbm.at[posb], sem_s).wait()
        for k in range(_BROWS // _LANE):
            posb[pl.ds(k * _LANE, _LANE)] = jnp.full(
                (_LANE,), _DUMP, jnp.int32
            )

    def process_pair(g0, bcount):
        pairid = g0 >> 1

        def scan_hits(t, bc):
            v = clist[pl.ds(t * _LANE, _LANE)]
            pv = plist[pl.ds(t * _LANE, _LANE)]
            valid = (t * _LANE + iota) < cnt
            m0 = ((v >> 9) == pairid) & valid
            n = plsc.all_reduce_population_count(m0)[0]

            def hit(h, carry):
                m, bc2 = carry
                ps = plsc.cumsum(m.astype(jnp.int32))
                onehot = m & (ps == 1)
                zero = jnp.zeros((_LANE,), jnp.int32)
                c_h = jnp.sum(jnp.where(onehot, v, zero))
                p_h = jnp.sum(jnp.where(onehot, pv, zero))
                g_h = c_h >> 8
                c_loc = (c_h & 255) + jnp.where(
                    g_h == _NCHUNK - 1, 128, 0
                )
                slot16 = jnp.full((_LANE,), g_h & (_NSLOT - 1), jnp.int32)
                loc16 = jnp.full((_LANE,), c_loc, jnp.int32)
                brow = jnp.full((_LANE,), bc2, jnp.int32)
                for k in range(EMBED_DIM // _LANE):
                    piece = plsc.load_gather(
                        cbuf, [slot16, iota + k * _LANE, loc16]
                    )
                    plsc.store_scatter(
                        batch, [brow, k * _LANE + iota], piece
                    )
                plsc.store_scatter(
                    posb,
                    [brow],
                    jnp.full((_LANE,), p_h, jnp.int32),
                    mask=iota == 0,
                )
                do_flush = bc2 + 1 == _BROWS

                @pl.when(do_flush)
                def _():
                    flush()

                bc3 = jnp.where(do_flush, 0, bc2 + 1)
                return m & (~onehot), bc3

            _, bc_out = lax.fori_loop(0, n, hit, (m0, bc))
            return bc_out

        return lax.fori_loop(0, nvec, scan_hits, bcount)

    for s in range(2):

        @pl.when(wlo + s < whi)
        def _():
            _issue(table_t, cbuf, sem_c, wlo + s, s)

    def outer(p2, bcount):
        for pp in range(2):
            g0 = wlo + (p2 * 2 + pp) * 2
            s0 = (pp * 2) % _NSLOT
            for s in range(2):

                @pl.when(g0 + 2 + s < whi)
                def _():
                    _issue(
                        table_t, cbuf, sem_c, g0 + 2 + s,
                        (s0 + 2 + s) % _NSLOT,
                    )

            for s in range(2):

                @pl.when(g0 + s < whi)
                def _():
                    _wait_chunk(table_t, cbuf, sem_c, s0 + s)

            bcount = lax.cond(
                g0 < whi,
                lambda bc: process_pair(g0, bc),
                lambda bc: bc,
                bcount,
            )
        return bcount

    bfin = lax.fori_loop(0, _CPW // 4, outer, jnp.int32(0))

    @pl.when(bfin > 0)
    def _():
        flush()


@jax.jit
def kernel(nodes, embedding_weight):
    mesh = plsc.VectorSubcoreMesh(core_axis_name="c", subcore_axis_name="s")
    run = functools.partial(
        pl.kernel,
        mesh=mesh,
        out_type=jax.ShapeDtypeStruct((BATCH + 1, 128), jnp.float32),
        scratch_types=[
            pltpu.VMEM((BATCH,), jnp.int32),
            pltpu.VMEM((BATCH,), jnp.int32),
            pltpu.VMEM((BATCH,), jnp.int32),
            pltpu.VMEM((_NSLOT, EMBED_DIM, _CCOLS), jnp.float32),
            pltpu.VMEM((_BROWS, 128), jnp.float32),
            pltpu.VMEM((_BROWS,), jnp.int32),
            pltpu.SemaphoreType.DMA,
            pltpu.SemaphoreType.DMA,
        ],
        compiler_params=pltpu.CompilerParams(
            use_tc_tiling_on_sc=True, needs_layout_passes=False
        ),
    )(_body)
    padded = run(embedding_weight.T, nodes.astype(jnp.int32))
    return padded[:BATCH, :EMBED_DIM]


# speedup vs baseline: 3.2434x; 1.0268x over previous
import functools

import jax
import jax.numpy as jnp
from jax import lax
from jax.experimental import pallas as pl
from jax.experimental.pallas import tpu as pltpu
from jax.experimental.pallas import tpu_sc as plsc

USER_NUM = 1000000
EMBED_DIM = 64
BATCH = 16384

_NC = 2
_NS = 16
_NW = _NC * _NS
_LANE = 16
_CCOLS = 256
_NCHUNK = -(-USER_NUM // _CCOLS)
_CPW = 124
_MAXOFF = USER_NUM - 192
_NSLOT = 4
_BROWS = 64
_DUMP = BATCH
_IB = "promise_in_bounds"


def _issue(table_t, cbuf, sem, g, slot):
    coff = pl.multiple_of(jnp.minimum(g * _CCOLS, _MAXOFF), 128)
    return pltpu.async_copy(
        table_t.at[:, pl.ds(coff, _CCOLS)], cbuf.at[slot], sem
    )


def _wait_chunk(table_t, cbuf, sem, slot):
    pltpu.make_async_copy(
        table_t.at[:, pl.ds(0, _CCOLS)], cbuf.at[slot], sem
    ).wait()


def _body(table_t, idx_hbm, out_hbm, clist, plist, cbuf, batch, posb,
          sem_c, sem_s):
    wid = lax.axis_index("s") * _NC + lax.axis_index("c")
    wlo = wid * _CPW
    whi = jnp.minimum(wlo + _CPW, _NCHUNK)
    clo = wlo * _CCOLS
    chi = jnp.minimum(whi * _CCOLS, USER_NUM)

    pltpu.sync_copy(idx_hbm, clist)

    iota = lax.iota(jnp.int32, _LANE)

    def scan_in(t, cnt):
        v = clist[pl.ds(t * _LANE, _LANE)]
        m = (v >= clo) & (v < chi)
        ps = plsc.cumsum(m.astype(jnp.int32))
        tgt = cnt + ps - 1
        plsc.store_scatter(clist, [tgt], v, mask=m)
        plsc.store_scatter(plist, [tgt], t * _LANE + iota, mask=m)
        return cnt + ps[_LANE - 1]

    cnt = lax.fori_loop(0, BATCH // _LANE, scan_in, jnp.int32(0))
    nvec = (cnt + _LANE - 1) // _LANE

    dump16 = jnp.full((_LANE,), _DUMP, jnp.int32)
    for par in range(2):
        for k in range(_BROWS // _LANE):
            plsc.store_scatter(
                posb, [jnp.full((_LANE,), par, jnp.int32), k * _LANE + iota],
                dump16,
            )

    def drain_scatter():
        pltpu.make_async_copy(
            batch.at[0], out_hbm.at[posb.at[0]], sem_s
        ).wait()

    def process_pair(g0, state):
        pairid = g0 >> 1

        def scan_hits(t, st):
            v = clist[pl.ds(t * _LANE, _LANE)]
            pv = plist[pl.ds(t * _LANE, _LANE)]
            valid = (t * _LANE + iota) < cnt
            m0 = ((v >> 9) == pairid) & valid
            n = plsc.cumsum(m0.astype(jnp.int32))[_LANE - 1]

            def hit(h, carry):
                m, bc, fb, pend = carry
                lanes = plsc.all_reduce_ffs(m)
                c16 = v.at[lanes].get(mode=_IB)
                p16 = pv.at[lanes].get(mode=_IB)
                loc16 = (c16 & 255) + jnp.where(
                    (c16 >> 8) == _NCHUNK - 1, 128, 0
                )
                slot16 = (c16 >> 8) & (_NSLOT - 1)
                fb16 = jnp.full((_LANE,), fb, jnp.int32)
                brow = jnp.full((_LANE,), bc, jnp.int32)
                for k in range(EMBED_DIM // _LANE):
                    piece = plsc.load_gather(
                        cbuf, [slot16, iota + k * _LANE, loc16]
                    )
                    plsc.store_scatter(
                        batch, [fb16, brow, k * _LANE + iota], piece
                    )
                plsc.store_scatter(
                    posb, [fb16, brow], p16, mask=iota == 0
                )
                do_flush = bc + 1 == _BROWS

                @pl.when(do_flush)
                def _():
                    @pl.when(pend == 1)
                    def _():
                        drain_scatter()

                    pltpu.async_copy(
                        batch.at[fb], out_hbm.at[posb.at[fb]], sem_s
                    )
                    nfb16 = jnp.full((_LANE,), 1 - fb, jnp.int32)
                    for k in range(_BROWS // _LANE):
                        plsc.store_scatter(
                            posb, [nfb16, k * _LANE + iota], dump16
                        )

                bc2 = jnp.where(do_flush, 0, bc + 1)
                fb2 = jnp.where(do_flush, 1 - fb, fb)
                pend2 = jnp.where(do_flush, 1, pend)
                return m & (iota != lanes), bc2, fb2, pend2

            out = lax.fori_loop(0, n, hit, (m0,) + st)
            return out[1:]

        return lax.fori_loop(0, nvec, scan_hits, state)

    for s in range(2):

        @pl.when(wlo + s < whi)
        def _():
            _issue(table_t, cbuf, sem_c, wlo + s, s)

    def outer(p2, state):
        for pp in range(2):
            g0 = wlo + (p2 * 2 + pp) * 2
            s0 = (pp * 2) % _NSLOT
            for s in range(2):

                @pl.when(g0 + 2 + s < whi)
                def _():
                    _issue(
                        table_t, cbuf, sem_c, g0 + 2 + s,
                        (s0 + 2 + s) % _NSLOT,
                    )

            for s in range(2):

                @pl.when(g0 + s < whi)
                def _():
                    _wait_chunk(table_t, cbuf, sem_c, s0 + s)

            state = lax.cond(
                g0 < whi,
                lambda st: process_pair(g0, st),
                lambda st: st,
                state,
            )
        return state

    bfin, ffin, pfin = lax.fori_loop(
        0, _CPW // 4, outer, (jnp.int32(0), jnp.int32(0), jnp.int32(0))
    )

    @pl.when(pfin == 1)
    def _():
        drain_scatter()

    @pl.when(bfin > 0)
    def _():
        pltpu.async_copy(
            batch.at[ffin], out_hbm.at[posb.at[ffin]], sem_s
        )
        drain_scatter()


@jax.jit
def kernel(nodes, embedding_weight):
    mesh = plsc.VectorSubcoreMesh(core_axis_name="c", subcore_axis_name="s")
    run = functools.partial(
        pl.kernel,
        mesh=mesh,
        out_type=jax.ShapeDtypeStruct((BATCH + 1, 128), jnp.float32),
        scratch_types=[
            pltpu.VMEM((BATCH,), jnp.int32),
            pltpu.VMEM((BATCH,), jnp.int32),
            pltpu.VMEM((_NSLOT, EMBED_DIM, _CCOLS), jnp.float32),
            pltpu.VMEM((2, _BROWS, 128), jnp.float32),
            pltpu.VMEM((2, _BROWS), jnp.int32),
            pltpu.SemaphoreType.DMA,
            pltpu.SemaphoreType.DMA,
        ],
        compiler_params=pltpu.CompilerParams(
            use_tc_tiling_on_sc=True, needs_layout_passes=False
        ),
    )(_body)
    padded = run(embedding_weight.T, nodes.astype(jnp.int32))
    return padded[:BATCH, :EMBED_DIM]
